# baseline (speedup 1.0000x reference)
import functools

import jax
import jax.numpy as jnp
from jax import lax
from jax.experimental import pallas as pl
from jax.experimental.pallas import tpu as pltpu
from jax.experimental.pallas import tpu_sc as plsc

B = 16384
K = 32
NC = 2
NS = 16
NW = NC * NS
BPW = B // NW
CHUNK = 128
NCH = BPW // CHUNK
LANES = 16
GROUPS = BPW // LANES


def _body(u_emb, m_emb, u_bias, m_bias, u_idx, m_idx, out,
          uidx_v, midx_v, u_rows, m_rows, ub_v, mb_v, out_v, sem):
    wid = lax.axis_index("s") * NC + lax.axis_index("c")

    pltpu.sync_copy(u_idx.at[wid], uidx_v)
    pltpu.sync_copy(m_idx.at[wid], midx_v)

    copies = []
    for j in range(NCH):
        sl = pl.ds(j * CHUNK, CHUNK)
        copies.append(pltpu.async_copy(u_emb.at[uidx_v.at[j]], u_rows.at[sl], sem))
        copies.append(pltpu.async_copy(m_emb.at[midx_v.at[j]], m_rows.at[sl], sem))
        copies.append(pltpu.async_copy(u_bias.at[uidx_v.at[j]], ub_v.at[sl], sem))
        copies.append(pltpu.async_copy(m_bias.at[midx_v.at[j]], mb_v.at[sl], sem))
    for c in copies:
        c.wait()

    def group_body(g, carry):
        base = g * LANES
        lane = base + lax.iota(jnp.int32, LANES)
        acc = ub_v[pl.ds(base, LANES)] + mb_v[pl.ds(base, LANES)]
        for k in range(K):
            kv = jnp.full((LANES,), k, jnp.int32)
            gu = plsc.load_gather(u_rows, [lane, kv])
            gm = plsc.load_gather(m_rows, [lane, kv])
            acc = acc + gu * gm
        out_v[pl.ds(base, LANES)] = acc
        return carry

    lax.fori_loop(0, GROUPS, group_body, 0)
    pltpu.sync_copy(out_v, out.at[pl.ds(wid * BPW, BPW)])


_run = functools.partial(
    pl.kernel,
    out_type=jax.ShapeDtypeStruct((B,), jnp.float32),
    mesh=plsc.VectorSubcoreMesh(core_axis_name="c", subcore_axis_name="s"),
    compiler_params=pltpu.CompilerParams(
        needs_layout_passes=False, use_tc_tiling_on_sc=False),
    scratch_types=[
        pltpu.VMEM((NCH, CHUNK), jnp.int32),
        pltpu.VMEM((NCH, CHUNK), jnp.int32),
        pltpu.VMEM((BPW, K), jnp.float32),
        pltpu.VMEM((BPW, K), jnp.float32),
        pltpu.VMEM((BPW,), jnp.float32),
        pltpu.VMEM((BPW,), jnp.float32),
        pltpu.VMEM((BPW,), jnp.float32),
        pltpu.SemaphoreType.DMA,
    ],
)(_body)


def kernel(x, u_embedding, m_embedding, u_bias, m_bias):
    u_idx = x[:, 0].astype(jnp.int32).reshape(NW, NCH, CHUNK)
    m_idx = x[:, 1].astype(jnp.int32).reshape(NW, NCH, CHUNK)
    return _run(u_embedding, m_embedding,
                u_bias.reshape(-1), m_bias.reshape(-1), u_idx, m_idx)
